# baseline (speedup 1.0000x reference)
import functools

import jax
import jax.numpy as jnp
from jax import lax
from jax.experimental import pallas as pl
from jax.experimental.pallas import tpu as pltpu
from jax.experimental.pallas import tpu_sc as plsc

B = 16384
MF_D = 8
MLP_D = 32


def _sc_gather(user, item, mf_user_table, mf_item_table,
               mlp_user_table, mlp_item_table):
  info = plsc.get_sparse_core_info()
  nw = info.num_cores * info.num_subcores
  b_per_w = B // nw
  ch = 128
  n_chunks = b_per_w // ch
  mesh = plsc.VectorSubcoreMesh(core_axis_name="c", subcore_axis_name="s")

  f32 = jnp.float32

  @functools.partial(
      pl.kernel,
      mesh=mesh,
      compiler_params=pltpu.CompilerParams(use_tc_tiling_on_sc=False),
      out_type=[
          jax.ShapeDtypeStruct((B, MF_D), f32),
          jax.ShapeDtypeStruct((B, MF_D), f32),
          jax.ShapeDtypeStruct((B, MLP_D), f32),
          jax.ShapeDtypeStruct((B, MLP_D), f32),
      ],
      scratch_types=[
          pltpu.VMEM((ch,), jnp.int32),
          pltpu.VMEM((ch,), jnp.int32),
          pltpu.VMEM((ch, MF_D), f32),
          pltpu.VMEM((ch, MF_D), f32),
          pltpu.VMEM((ch, MLP_D), f32),
          pltpu.VMEM((ch, MLP_D), f32),
          pltpu.SemaphoreType.DMA,
      ],
  )
  def gather_kernel(user_h, item_h, mfu_h, mfi_h, mlu_h, mli_h,
                    out_mfu, out_mfi, out_mlu, out_mli,
                    idx_u, idx_i, r_mfu, r_mfi, r_mlu, r_mli, sem):
    wid = lax.axis_index("s") * info.num_cores + lax.axis_index("c")
    base = wid * b_per_w
    for c in range(n_chunks):
      off = base + c * ch
      pltpu.sync_copy(user_h.at[pl.ds(off, ch)], idx_u)
      pltpu.sync_copy(item_h.at[pl.ds(off, ch)], idx_i)
      g1 = pltpu.async_copy(mfu_h.at[idx_u], r_mfu, sem)
      g2 = pltpu.async_copy(mfi_h.at[idx_i], r_mfi, sem)
      g3 = pltpu.async_copy(mlu_h.at[idx_u], r_mlu, sem)
      g4 = pltpu.async_copy(mli_h.at[idx_i], r_mli, sem)
      g1.wait()
      g2.wait()
      g3.wait()
      g4.wait()
      pltpu.sync_copy(r_mfu, out_mfu.at[pl.ds(off, ch)])
      pltpu.sync_copy(r_mfi, out_mfi.at[pl.ds(off, ch)])
      pltpu.sync_copy(r_mlu, out_mlu.at[pl.ds(off, ch)])
      pltpu.sync_copy(r_mli, out_mli.at[pl.ds(off, ch)])

  return gather_kernel(user, item, mf_user_table, mf_item_table,
                       mlp_user_table, mlp_item_table)


def _tc_mlp_body(mfu_ref, mfi_ref, mlu_ref, mli_ref,
                 w1u_ref, w1i_ref, b1_ref, w2_ref, b2_ref, w3_ref, b3_ref,
                 wl_mf_ref, wl_mlp_ref, bl_ref, out_ref):
  f32 = jnp.float32
  h = jnp.dot(mlu_ref[...], w1u_ref[...], preferred_element_type=f32)
  h = h + jnp.dot(mli_ref[...], w1i_ref[...], preferred_element_type=f32)
  h = jnp.maximum(h + b1_ref[...], 0.0)
  h = jnp.maximum(
      jnp.dot(h, w2_ref[...], preferred_element_type=f32) + b2_ref[...], 0.0)
  h = jnp.maximum(
      jnp.dot(h, w3_ref[...], preferred_element_type=f32) + b3_ref[...], 0.0)
  mf = mfu_ref[...] * mfi_ref[...]
  logit = (jnp.sum(mf * wl_mf_ref[...], axis=1)
           + jnp.sum(h * wl_mlp_ref[...], axis=1)
           + bl_ref[0, 0])
  out_ref[...] = jax.nn.sigmoid(logit)


def _tc_mlp(mfu, mfi, mlu, mli, W1, b1, W2, b2, W3, b3, Wl, bl):
  blk = 2048
  grid = (B // blk,)
  f32 = jnp.float32
  w1u = W1[:MLP_D]
  w1i = W1[MLP_D:]
  wl_mf = Wl[:MF_D, 0].reshape(1, MF_D)
  wl_mlp = Wl[MF_D:, 0].reshape(1, Wl.shape[0] - MF_D)
  b1r = b1.reshape(1, -1)
  b2r = b2.reshape(1, -1)
  b3r = b3.reshape(1, -1)
  blr = bl.reshape(1, 1)

  def rows_spec(d):
    return pl.BlockSpec((blk, d), lambda i: (i, 0))

  def full_spec(a):
    return pl.BlockSpec(a.shape, lambda i: tuple(0 for _ in a.shape))

  return pl.pallas_call(
      _tc_mlp_body,
      grid=grid,
      in_specs=[
          rows_spec(MF_D), rows_spec(MF_D), rows_spec(MLP_D), rows_spec(MLP_D),
          full_spec(w1u), full_spec(w1i), full_spec(b1r), full_spec(W2),
          full_spec(b2r), full_spec(W3), full_spec(b3r), full_spec(wl_mf),
          full_spec(wl_mlp), full_spec(blr),
      ],
      out_specs=pl.BlockSpec((blk,), lambda i: (i,)),
      out_shape=jax.ShapeDtypeStruct((B,), f32),
  )(mfu, mfi, mlu, mli, w1u, w1i, b1r, W2, b2r, W3, b3r, wl_mf, wl_mlp, blr)


def kernel(user, item, mf_user_table, mf_item_table, mlp_user_table,
           mlp_item_table, W1, b1, W2, b2, W3, b3, Wl, bl):
  user = user.astype(jnp.int32)
  item = item.astype(jnp.int32)
  mfu, mfi, mlu, mli = _sc_gather(user, item, mf_user_table, mf_item_table,
                                  mlp_user_table, mlp_item_table)
  return _tc_mlp(mfu, mfi, mlu, mli, W1, b1, W2, b2, W3, b3, Wl, bl)
